# Initial kernel scaffold; baseline (speedup 1.0000x reference)
import jax
import jax.numpy as jnp
from jax.experimental import pallas as pl


def kernel(data_pre, data_post, len_pre, len_post, emb_table, W, b):
    raise NotImplementedError("write your pallas kernel here")


# speedup vs baseline: 3.1949x; 3.1949x over previous
import functools

import jax
import jax.numpy as jnp
from jax import lax
from jax.experimental import pallas as pl
from jax.experimental.pallas import tpu as pltpu
from jax.experimental.pallas import tpu_sc as plsc

NC = 2
NS = 16
CHUNK = 128


def _sc_pool(idx_all, seg, table, B, L, D):
    per_w = (B * L) // NS
    n_chunks = per_w // CHUNK
    seg_w = B // NS

    mesh = plsc.VectorSubcoreMesh(core_axis_name="c", subcore_axis_name="s",
                                  num_cores=NC, num_subcores=NS)

    @functools.partial(
        pl.kernel,
        out_type=jax.ShapeDtypeStruct((2 * B, D), jnp.float32),
        mesh=mesh,
        scratch_types=dict(
            idx0=pltpu.VMEM((CHUNK,), jnp.int32),
            idx1=pltpu.VMEM((CHUNK,), jnp.int32),
            seg0=pltpu.VMEM((CHUNK,), jnp.int32),
            seg1=pltpu.VMEM((CHUNK,), jnp.int32),
            rows0=pltpu.VMEM((CHUNK, D), jnp.float32),
            rows1=pltpu.VMEM((CHUNK, D), jnp.float32),
            acc=pltpu.VMEM_SHARED((B, D), jnp.float32),
            sem_i0=pltpu.SemaphoreType.DMA,
            sem_i1=pltpu.SemaphoreType.DMA,
            sem_s0=pltpu.SemaphoreType.DMA,
            sem_s1=pltpu.SemaphoreType.DMA,
            sem_g0=pltpu.SemaphoreType.DMA,
            sem_g1=pltpu.SemaphoreType.DMA,
        ),
        compiler_params=pltpu.CompilerParams(use_tc_tiling_on_sc=False),
    )
    def body(idx_hbm, seg_hbm, table_hbm, out_hbm, *, idx0, idx1, seg0, seg1,
             rows0, rows1, acc, sem_i0, sem_i1, sem_s0, sem_s1, sem_g0, sem_g1):
        c = lax.axis_index("c")
        s = lax.axis_index("s")
        base = c * (B * L) + s * per_w
        sbase = s * per_w

        idxb = (idx0, idx1)
        segb = (seg0, seg1)
        rowsb = (rows0, rows1)
        sem_i = (sem_i0, sem_i1)
        sem_s = (sem_s0, sem_s1)
        sem_g = (sem_g0, sem_g1)

        zv = jnp.zeros((16,), jnp.float32)

        def zrow(i, _):
            for h in range(D // 16):
                rows0[i, pl.ds(h * 16, 16)] = zv
            return 0

        lax.fori_loop(0, CHUNK, zrow, 0)
        for j in range(seg_w // CHUNK):
            pltpu.sync_copy(rows0,
                            acc.at[pl.ds(s * seg_w + j * CHUNK, CHUNK)])

        def start_a(k, b):
            pltpu.async_copy(idx_hbm.at[pl.ds(base + k * CHUNK, CHUNK)],
                             idxb[b], sem_i[b])
            pltpu.async_copy(seg_hbm.at[pl.ds(sbase + k * CHUNK, CHUNK)],
                             segb[b], sem_s[b])

        def wait_a(k, b):
            pltpu.make_async_copy(idx_hbm.at[pl.ds(base + k * CHUNK, CHUNK)],
                                  idxb[b], sem_i[b]).wait()
            pltpu.make_async_copy(seg_hbm.at[pl.ds(sbase + k * CHUNK, CHUNK)],
                                  segb[b], sem_s[b]).wait()

        def start_g(b):
            pltpu.async_copy(table_hbm.at[idxb[b]], rowsb[b], sem_g[b])

        def wait_g(b):
            pltpu.make_async_copy(table_hbm.at[idxb[b]], rowsb[b],
                                  sem_g[b]).wait()

        start_a(0, 0)
        start_a(1, 1)
        wait_a(0, 0)
        start_g(0)

        def step(k, _):
            for b in range(2):
                kk = k + b
                wait_g(b)
                @pl.when(kk + 1 < n_chunks)
                def _():
                    wait_a(kk + 1, 1 - b)
                    start_g(1 - b)

                pltpu.sync_copy(rowsb[b], acc.at[segb[b]], add=True)

                @pl.when(kk + 2 < n_chunks)
                def _():
                    start_a(kk + 2, b)

            return 0

        lax.fori_loop(0, n_chunks // 2, lambda k, x: step(2 * k, x), 0,
                      unroll=False)

        pltpu.sync_copy(acc.at[pl.ds(s * seg_w, seg_w)],
                        out_hbm.at[pl.ds(c * B + s * seg_w, seg_w)])

    return body(idx_all, seg, table)


def _tc_head(sums_pre, sums_post, len_pre, len_post, W, b, B, D, C, bm=2048):
    def body(sp_ref, ss_ref, lp_ref, ls_ref, w_ref, b_ref, o_ref):
        lp = lp_ref[...].astype(jnp.float32)
        ls = ls_ref[...].astype(jnp.float32)
        h = jnp.concatenate([sp_ref[...] / lp, ss_ref[...] / ls], axis=1)
        w = w_ref[...]
        logits = lax.dot_general(h, w, (((1,), (1,)), ((), ())),
                                 preferred_element_type=jnp.float32)
        logits = logits + b_ref[...]
        m = jnp.max(logits, axis=1, keepdims=True)
        e = jnp.exp(logits - m)
        lse = jnp.log(jnp.sum(e, axis=1, keepdims=True)) + m
        o_ref[...] = logits - lse

    grid = (B // bm,)
    return pl.pallas_call(
        body,
        grid=grid,
        in_specs=[
            pl.BlockSpec((bm, D), lambda i: (i, 0)),
            pl.BlockSpec((bm, D), lambda i: (i, 0)),
            pl.BlockSpec((bm, 1), lambda i: (i, 0)),
            pl.BlockSpec((bm, 1), lambda i: (i, 0)),
            pl.BlockSpec((C, 2 * D), lambda i: (0, 0)),
            pl.BlockSpec((1, C), lambda i: (0, 0)),
        ],
        out_specs=pl.BlockSpec((bm, C), lambda i: (i, 0)),
        out_shape=jax.ShapeDtypeStruct((B, C), jnp.float32),
    )(sums_pre, sums_post, len_pre.reshape(B, 1), len_post.reshape(B, 1),
      W, b.reshape(1, C))


def kernel(data_pre, data_post, len_pre, len_post, emb_table, W, b):
    B, L = data_pre.shape
    V, D = emb_table.shape
    C = W.shape[0]
    idx_all = jnp.concatenate([data_pre.reshape(-1), data_post.reshape(-1)])
    idx_all = idx_all.astype(jnp.int32)
    seg = (jnp.arange(B * L, dtype=jnp.int32) // L).astype(jnp.int32)
    sums = _sc_pool(idx_all, seg, emb_table, B, L, D)
    return _tc_head(sums[:B], sums[B:], len_pre, len_post, W, b, B, D, C)
